# baseline (speedup 1.0000x reference)
import functools

import jax
import jax.numpy as jnp
from jax import lax
from jax.experimental import pallas as pl
from jax.experimental.pallas import tpu as pltpu
from jax.experimental.pallas import tpu_sc as plsc

N = 4096
D = 512
C = 128
BLK = 256
NBLK = N // BLK
TW = 640
ALPHA = 0.2
_NT = (((1,), (1,)), ((), ()))

_NC = 2
_NS = 16
_NW = _NC * _NS
_RPW = N // _NW
_CH = 32
_NCHUNK = _RPW // _CH


def _bf16(x):
    return x.astype(jnp.bfloat16)


def _proj_body(x_ref, enc_ref, watt_ref, a8_ref, xproj_ref, wh_ref, e_ref):
    xp = jnp.dot(x_ref[...], enc_ref[...], preferred_element_type=jnp.float32)
    wh = jnp.dot(_bf16(xp), watt_ref[...], preferred_element_type=jnp.float32)
    e_ref[...] = lax.dot_general(wh, a8_ref[...], _NT,
                                 precision=lax.Precision.HIGHEST,
                                 preferred_element_type=jnp.float32)
    xproj_ref[...] = _bf16(xp)
    wh_ref[...] = _bf16(wh)


def _rank_body(e2row_ref, e_ref, idx_ref):
    i = pl.program_id(0)
    row = e2row_ref[...]
    e1b = e_ref[:, 0:1]
    e2b = e_ref[:, 1:2]
    jglob = (lax.broadcasted_iota(jnp.int32, (BLK, 1), 0) + i * BLK)
    lane = lax.broadcasted_iota(jnp.int32, (BLK, N), 1)
    less = (row < e2b).astype(jnp.int32)
    tie = jnp.logical_and(row == e2b, lane < jglob).astype(jnp.int32)
    rank = jnp.sum(less + tie, axis=1, keepdims=True)
    k = jnp.sum((row <= -e1b).astype(jnp.int32), axis=1, keepdims=True)
    kg = jnp.clip(k - 1, 0, N - 1)
    kh = jnp.clip(k, 0, N - 1)
    zero = jnp.zeros((BLK, 4), jnp.int32)
    idx_ref[...] = jnp.concatenate([rank, k, kg, kh, zero], axis=1)


def _table_body(e2row_ref, e_ref, wh_ref, tu_ref, tv_ref):
    m2 = jnp.max(e2row_ref[...])
    e2b = e_ref[:, 1:2]
    u = jnp.exp(e2b - m2)
    v = jnp.exp(ALPHA * (e2b - m2))
    whf = wh_ref[...].astype(jnp.float32)
    pad = jnp.zeros((BLK, TW - D - 1), jnp.float32)
    tu_ref[...] = jnp.concatenate([u * whf, u, pad], axis=1)
    tv_ref[...] = jnp.concatenate([v * whf, v, pad], axis=1)


def _cumsum_body(su_ref, sv_ref, cu_ref, cv_ref, ucar, vcar):
    i = pl.program_id(0)

    @pl.when(i == 0)
    def _():
        ucar[...] = jnp.zeros((1, TW), jnp.float32)
        vcar[...] = jnp.zeros((1, TW), jnp.float32)

    r = lax.broadcasted_iota(jnp.int32, (BLK, BLK), 0)
    c = lax.broadcasted_iota(jnp.int32, (BLK, BLK), 1)
    lv = (r >= c).astype(jnp.float32)
    lu = (r <= c).astype(jnp.float32)
    cv = jnp.dot(lv, sv_ref[...], precision=lax.Precision.HIGHEST,
                 preferred_element_type=jnp.float32) + vcar[...]
    cu = jnp.dot(lu, su_ref[...], precision=lax.Precision.HIGHEST,
                 preferred_element_type=jnp.float32) + ucar[...]
    cv_ref[...] = cv
    cu_ref[...] = cu
    vcar[...] = cv[BLK - 1:BLK, :]
    ucar[...] = cu[0:1, :]


def _combine_body(e2row_ref, e_ref, idx_ref, gg_ref, hg_ref, xproj_ref,
                  w1_ref, z1_ref):
    m2 = jnp.max(e2row_ref[...])
    e1b = e_ref[:, 0:1]
    k = idx_ref[:, 1:2]
    t = e1b + m2
    m = jnp.where(t >= 0, t, ALPHA * t)
    p = jnp.exp(t - m)
    q = jnp.exp(ALPHA * t - m)
    mg = (k > 0).astype(jnp.float32)
    mh = (k < N).astype(jnp.float32)
    g = gg_ref[:, 0:D] * mg
    gs = gg_ref[:, D:D + 1] * mg
    h = hg_ref[:, 0:D] * mh
    hs = hg_ref[:, D:D + 1] * mh
    num = p * h + q * g
    z = p * hs + q * gs
    x_att = num / z
    x_ent = xproj_ref[...].astype(jnp.float32) * x_att
    z1_ref[...] = _bf16(
        jnp.dot(_bf16(x_ent), w1_ref[...], preferred_element_type=jnp.float32))


def _gcn1_body(adj_ref, z1_ref, b1_ref, w2_ref, z2_ref):
    y1 = jnp.dot(adj_ref[...], z1_ref[...], preferred_element_type=jnp.float32)
    y1 = jnp.maximum(y1 + b1_ref[...], 0.0)
    z2_ref[...] = _bf16(
        jnp.dot(_bf16(y1), w2_ref[...], preferred_element_type=jnp.float32))


def _gcn2_body(adj_ref, z2_ref, b2_ref, w3_ref, z3_ref):
    y2 = jnp.dot(adj_ref[...], z2_ref[...], preferred_element_type=jnp.float32)
    y2 = y2 + b2_ref[...]
    z3_ref[...] = _bf16(
        jnp.dot(_bf16(y2), w3_ref[...], preferred_element_type=jnp.float32))


def _gcn3_body(adj_ref, z3_ref, b3_ref, out_ref):
    y3 = jnp.dot(adj_ref[...], z3_ref[...], preferred_element_type=jnp.float32)
    y3 = y3 + b3_ref[...]
    m = jnp.max(y3, axis=1, keepdims=True)
    s = y3 - m
    lse = jnp.log(jnp.sum(jnp.exp(s), axis=1, keepdims=True))
    out_ref[...] = s - lse


def _row_blocked(d):
    return pl.BlockSpec((BLK, d), lambda i: (i, 0))


def _whole(r, c_):
    return pl.BlockSpec((r, c_), lambda i: (0, 0))


def _sc_scatter_body(tu, tv, rank1d, su, sv, idx_v, buf_u, buf_v, sem_u,
                     sem_v):
    wid = lax.axis_index("s") * _NC + lax.axis_index("c")
    base = wid * _RPW
    for ci in range(_NCHUNK):
        off = base + ci * _CH
        pltpu.sync_copy(rank1d.at[pl.ds(off, _CH)], idx_v)
        pltpu.sync_copy(tu.at[pl.ds(off, _CH)], buf_u)
        pltpu.sync_copy(tv.at[pl.ds(off, _CH)], buf_v)
        pltpu.async_copy(buf_u, su.at[idx_v], sem_u).wait()
        pltpu.async_copy(buf_v, sv.at[idx_v], sem_v).wait()


def _sc_gather_body(cu, cv, kg1d, kh1d, hg, gg, idx_v, buf_u, buf_v, sem_u,
                    sem_v):
    wid = lax.axis_index("s") * _NC + lax.axis_index("c")
    base = wid * _RPW
    for ci in range(_NCHUNK):
        off = base + ci * _CH
        pltpu.sync_copy(kh1d.at[pl.ds(off, _CH)], idx_v)
        pltpu.async_copy(cu.at[idx_v], buf_u, sem_u).wait()
        pltpu.sync_copy(buf_u, hg.at[pl.ds(off, _CH)])
        pltpu.sync_copy(kg1d.at[pl.ds(off, _CH)], idx_v)
        pltpu.async_copy(cv.at[idx_v], buf_v, sem_v).wait()
        pltpu.sync_copy(buf_v, gg.at[pl.ds(off, _CH)])


def _sc_mesh():
    return plsc.VectorSubcoreMesh(core_axis_name="c", subcore_axis_name="s")


def _sc_scatter(tu, tv, rank1d):
    f = pl.kernel(
        _sc_scatter_body,
        mesh=_sc_mesh(),
        out_type=[jax.ShapeDtypeStruct((N, TW), jnp.float32)] * 2,
        scratch_types=[
            pltpu.VMEM((_CH,), jnp.int32),
            pltpu.VMEM((_CH, TW), jnp.float32),
            pltpu.VMEM((_CH, TW), jnp.float32),
            pltpu.SemaphoreType.DMA,
            pltpu.SemaphoreType.DMA,
        ],
    )
    return f(tu, tv, rank1d)


def _sc_gather(cu, cv, kg1d, kh1d):
    f = pl.kernel(
        _sc_gather_body,
        mesh=_sc_mesh(),
        out_type=[jax.ShapeDtypeStruct((N, TW), jnp.float32)] * 2,
        scratch_types=[
            pltpu.VMEM((_CH,), jnp.int32),
            pltpu.VMEM((_CH, TW), jnp.float32),
            pltpu.VMEM((_CH, TW), jnp.float32),
            pltpu.SemaphoreType.DMA,
            pltpu.SemaphoreType.DMA,
        ],
    )
    return f(cu, cv, kg1d, kh1d)


def kernel(x_org, adj, encoder1, W_att, a_att, gc1_W, gc1_b, gc2_W, gc2_b,
           gc3_W, gc3_b):
    grid = (NBLK,)
    adjb = _bf16(adj)
    a8 = jnp.zeros((8, D), jnp.float32).at[0:2].set(a_att.reshape(2, D))

    xproj, wh, e = pl.pallas_call(
        _proj_body,
        grid=grid,
        in_specs=[_row_blocked(D), _whole(D, D), _whole(D, D), _whole(8, D)],
        out_specs=[_row_blocked(D), _row_blocked(D), _row_blocked(8)],
        out_shape=[
            jax.ShapeDtypeStruct((N, D), jnp.bfloat16),
            jax.ShapeDtypeStruct((N, D), jnp.bfloat16),
            jax.ShapeDtypeStruct((N, 8), jnp.float32),
        ],
    )(_bf16(x_org), _bf16(encoder1), _bf16(W_att), a8)

    e2row = e[:, 1:2].reshape(1, N)

    idx = pl.pallas_call(
        _rank_body,
        grid=grid,
        in_specs=[_whole(1, N), _row_blocked(8)],
        out_specs=_row_blocked(8),
        out_shape=jax.ShapeDtypeStruct((N, 8), jnp.int32),
    )(e2row, e)

    tu, tv = pl.pallas_call(
        _table_body,
        grid=grid,
        in_specs=[_whole(1, N), _row_blocked(8), _row_blocked(D)],
        out_specs=[_row_blocked(TW), _row_blocked(TW)],
        out_shape=[jax.ShapeDtypeStruct((N, TW), jnp.float32)] * 2,
    )(e2row, e, wh)

    rank1d = idx[:, 0].reshape(N)
    kg1d = idx[:, 2].reshape(N)
    kh1d = idx[:, 3].reshape(N)

    su, sv = _sc_scatter(tu, tv, rank1d)

    cu, cv = pl.pallas_call(
        _cumsum_body,
        grid=grid,
        in_specs=[
            pl.BlockSpec((BLK, TW), lambda i: (NBLK - 1 - i, 0)),
            pl.BlockSpec((BLK, TW), lambda i: (i, 0)),
        ],
        out_specs=[
            pl.BlockSpec((BLK, TW), lambda i: (NBLK - 1 - i, 0)),
            pl.BlockSpec((BLK, TW), lambda i: (i, 0)),
        ],
        out_shape=[jax.ShapeDtypeStruct((N, TW), jnp.float32)] * 2,
        scratch_shapes=[
            pltpu.VMEM((1, TW), jnp.float32),
            pltpu.VMEM((1, TW), jnp.float32),
        ],
    )(su, sv)

    hg, gg = _sc_gather(cu, cv, kg1d, kh1d)

    z1 = pl.pallas_call(
        _combine_body,
        grid=grid,
        in_specs=[_whole(1, N), _row_blocked(8), _row_blocked(8),
                  _row_blocked(TW), _row_blocked(TW), _row_blocked(D),
                  _whole(D, D)],
        out_specs=_row_blocked(D),
        out_shape=jax.ShapeDtypeStruct((N, D), jnp.bfloat16),
    )(e2row, e, idx, gg, hg, xproj, _bf16(gc1_W))

    z2 = pl.pallas_call(
        _gcn1_body,
        grid=grid,
        in_specs=[_row_blocked(N), _whole(N, D), _whole(1, D), _whole(D, D)],
        out_specs=_row_blocked(D),
        out_shape=jax.ShapeDtypeStruct((N, D), jnp.bfloat16),
    )(adjb, z1, gc1_b.reshape(1, D), _bf16(gc2_W))

    z3 = pl.pallas_call(
        _gcn2_body,
        grid=grid,
        in_specs=[_row_blocked(N), _whole(N, D), _whole(1, D), _whole(D, C)],
        out_specs=_row_blocked(C),
        out_shape=jax.ShapeDtypeStruct((N, C), jnp.bfloat16),
    )(adjb, z2, gc2_b.reshape(1, D), _bf16(gc3_W))

    out = pl.pallas_call(
        _gcn3_body,
        grid=grid,
        in_specs=[_row_blocked(N), _whole(N, C), _whole(1, C)],
        out_specs=_row_blocked(C),
        out_shape=jax.ShapeDtypeStruct((N, C), jnp.float32),
    )(adjb, z3, gc3_b.reshape(1, C))

    return out


# speedup vs baseline: 1.4580x; 1.4580x over previous
import jax
import jax.numpy as jnp
from jax import lax
from jax.experimental import pallas as pl

N = 4096
D = 512
C = 128
BLK = 256
ALPHA = 0.2
_NT = (((1,), (1,)), ((), ()))


def _bf16(x):
    return x.astype(jnp.bfloat16)


def _proj_body(x_ref, enc_ref, watt_ref, xproj_ref, wh_ref):
    xp = jnp.dot(x_ref[...], enc_ref[...], preferred_element_type=jnp.float32)
    wh = jnp.dot(_bf16(xp), watt_ref[...], preferred_element_type=jnp.float32)
    xproj_ref[...] = _bf16(xp)
    wh_ref[...] = _bf16(wh)


def _attn_body(a_ref, wh_ref, xproj_ref, w1_ref, z1_ref):
    i = pl.program_id(0)
    wh = wh_ref[...]
    wh_blk = wh_ref[pl.ds(i * BLK, BLK), :]
    a8 = a_ref[...]
    eblk = lax.dot_general(wh_blk, a8, _NT, preferred_element_type=jnp.float32)
    erow = lax.dot_general(a8, wh, _NT, preferred_element_type=jnp.float32)
    e1 = eblk[:, 0:1]
    e2 = erow[1:2, :]
    m2 = jnp.max(e2)
    t = e1 + m2
    m = jnp.where(t >= 0, t, ALPHA * t)
    p = jnp.exp(t - m)
    q = jnp.exp(ALPHA * t - m)
    u = jnp.exp(e2 - m2)
    v = jnp.exp(ALPHA * (e2 - m2))
    s = jnp.where(e1 + e2 > 0, p * u, q * v)
    z = jnp.sum(s, axis=1, keepdims=True)
    acc = jnp.dot(_bf16(s), wh, preferred_element_type=jnp.float32)
    x_ent = xproj_ref[...].astype(jnp.float32) * (acc / z)
    z1_ref[...] = _bf16(
        jnp.dot(_bf16(x_ent), w1_ref[...], preferred_element_type=jnp.float32))


def _gcn1_body(adj_ref, z1_ref, b1_ref, w2_ref, z2_ref):
    y1 = jnp.dot(adj_ref[...], z1_ref[...], preferred_element_type=jnp.float32)
    y1 = jnp.maximum(y1 + b1_ref[...], 0.0)
    z2_ref[...] = _bf16(
        jnp.dot(_bf16(y1), w2_ref[...], preferred_element_type=jnp.float32))


def _gcn2_body(adj_ref, z2_ref, b2_ref, w3_ref, z3_ref):
    y2 = jnp.dot(adj_ref[...], z2_ref[...], preferred_element_type=jnp.float32)
    y2 = y2 + b2_ref[...]
    z3_ref[...] = _bf16(
        jnp.dot(_bf16(y2), w3_ref[...], preferred_element_type=jnp.float32))


def _gcn3_body(adj_ref, z3_ref, b3_ref, out_ref):
    y3 = jnp.dot(adj_ref[...], z3_ref[...], preferred_element_type=jnp.float32)
    y3 = y3 + b3_ref[...]
    m = jnp.max(y3, axis=1, keepdims=True)
    s = y3 - m
    lse = jnp.log(jnp.sum(jnp.exp(s), axis=1, keepdims=True))
    out_ref[...] = s - lse


def _row_blocked(d):
    return pl.BlockSpec((BLK, d), lambda i: (i, 0))


def _whole(r, c):
    return pl.BlockSpec((r, c), lambda i: (0, 0))


def kernel(x_org, adj, encoder1, W_att, a_att, gc1_W, gc1_b, gc2_W, gc2_b,
           gc3_W, gc3_b):
    grid = (N // BLK,)
    xb = _bf16(x_org)
    adjb = _bf16(adj)
    a_pair = jnp.zeros((8, D), jnp.bfloat16).at[0:2].set(_bf16(a_att.reshape(2, D)))

    xproj, wh = pl.pallas_call(
        _proj_body,
        grid=grid,
        in_specs=[_row_blocked(D), _whole(D, D), _whole(D, D)],
        out_specs=[_row_blocked(D), _row_blocked(D)],
        out_shape=[jax.ShapeDtypeStruct((N, D), jnp.bfloat16)] * 2,
    )(xb, _bf16(encoder1), _bf16(W_att))

    z1 = pl.pallas_call(
        _attn_body,
        grid=grid,
        in_specs=[_whole(8, D), _whole(N, D), _row_blocked(D), _whole(D, D)],
        out_specs=_row_blocked(D),
        out_shape=jax.ShapeDtypeStruct((N, D), jnp.bfloat16),
    )(a_pair, wh, xproj, _bf16(gc1_W))

    z2 = pl.pallas_call(
        _gcn1_body,
        grid=grid,
        in_specs=[_row_blocked(N), _whole(N, D), _whole(1, D), _whole(D, D)],
        out_specs=_row_blocked(D),
        out_shape=jax.ShapeDtypeStruct((N, D), jnp.bfloat16),
    )(adjb, z1, gc1_b.reshape(1, D), _bf16(gc2_W))

    z3 = pl.pallas_call(
        _gcn2_body,
        grid=grid,
        in_specs=[_row_blocked(N), _whole(N, D), _whole(1, D), _whole(D, C)],
        out_specs=_row_blocked(C),
        out_shape=jax.ShapeDtypeStruct((N, C), jnp.bfloat16),
    )(adjb, z2, gc2_b.reshape(1, D), _bf16(gc3_W))

    out = pl.pallas_call(
        _gcn3_body,
        grid=grid,
        in_specs=[_row_blocked(N), _whole(N, C), _whole(1, C)],
        out_specs=_row_blocked(C),
        out_shape=jax.ShapeDtypeStruct((N, C), jnp.float32),
    )(adjb, z3, gc3_b.reshape(1, C))

    return out


# speedup vs baseline: 1.4631x; 1.0035x over previous
import jax
import jax.numpy as jnp
from jax import lax
from jax.experimental import pallas as pl
from jax.experimental.pallas import tpu as pltpu

N = 4096
D = 512
C = 128
BLK = 256
ALPHA = 0.2
_NT = (((1,), (1,)), ((), ()))


def _bf16(x):
    return x.astype(jnp.bfloat16)


def _proj_body(x_ref, enc_ref, watt_ref, xproj_ref, wh_ref):
    xp = jnp.dot(x_ref[...], enc_ref[...], preferred_element_type=jnp.float32)
    wh = jnp.dot(_bf16(xp), watt_ref[...], preferred_element_type=jnp.float32)
    xproj_ref[...] = _bf16(xp)
    wh_ref[...] = _bf16(wh)


def _attn_body(a_ref, wh_ref, xproj_ref, w1_ref, z1_ref):
    i = pl.program_id(0)
    wh = wh_ref[...]
    wh_blk = wh_ref[pl.ds(i * BLK, BLK), :]
    a8 = a_ref[...]
    eblk = lax.dot_general(wh_blk, a8, _NT, preferred_element_type=jnp.float32)
    erow = lax.dot_general(a8, wh, _NT, preferred_element_type=jnp.float32)
    e1 = eblk[:, 0:1]
    e2 = erow[1:2, :]
    m2 = jnp.max(e2)
    t = e1 + m2
    m = jnp.where(t >= 0, t, ALPHA * t)
    p = jnp.exp(t - m)
    q = jnp.exp(ALPHA * t - m)
    u = jnp.exp(e2 - m2)
    v = jnp.exp(ALPHA * (e2 - m2))
    s = jnp.where(e1 + e2 > 0, p * u, q * v)
    z = jnp.sum(s, axis=1, keepdims=True)
    acc = jnp.dot(_bf16(s), wh, preferred_element_type=jnp.float32)
    x_ent = xproj_ref[...].astype(jnp.float32) * (acc / z)
    z1_ref[...] = _bf16(
        jnp.dot(_bf16(x_ent), w1_ref[...], preferred_element_type=jnp.float32))


def _gcn1_body(adj_ref, z1_ref, b1_ref, w2_ref, z2_ref):
    y1 = jnp.dot(adj_ref[...], z1_ref[...], preferred_element_type=jnp.float32)
    y1 = jnp.maximum(y1 + b1_ref[...], 0.0)
    z2_ref[...] = _bf16(
        jnp.dot(_bf16(y1), w2_ref[...], preferred_element_type=jnp.float32))


def _gcn2_body(adj_ref, z2_ref, b2_ref, w3_ref, z3_ref):
    y2 = jnp.dot(adj_ref[...], z2_ref[...], preferred_element_type=jnp.float32)
    y2 = y2 + b2_ref[...]
    z3_ref[...] = _bf16(
        jnp.dot(_bf16(y2), w3_ref[...], preferred_element_type=jnp.float32))


def _gcn3_body(adj_ref, z3_ref, b3_ref, out_ref):
    y3 = jnp.dot(adj_ref[...], z3_ref[...], preferred_element_type=jnp.float32)
    y3 = y3 + b3_ref[...]
    m = jnp.max(y3, axis=1, keepdims=True)
    s = y3 - m
    lse = jnp.log(jnp.sum(jnp.exp(s), axis=1, keepdims=True))
    out_ref[...] = s - lse


def _row_blocked(d):
    return pl.BlockSpec((BLK, d), lambda i: (i, 0))


def _whole(r, c):
    return pl.BlockSpec((r, c), lambda i: (0, 0))


def kernel(x_org, adj, encoder1, W_att, a_att, gc1_W, gc1_b, gc2_W, gc2_b,
           gc3_W, gc3_b):
    grid = (N // BLK,)
    xb = _bf16(x_org)
    adjb = _bf16(adj)
    a_pair = jnp.zeros((8, D), jnp.bfloat16).at[0:2].set(_bf16(a_att.reshape(2, D)))

    xproj, wh = pl.pallas_call(
        _proj_body,
        grid=grid,
        compiler_params=pltpu.CompilerParams(dimension_semantics=("parallel",)),
        in_specs=[_row_blocked(D), _whole(D, D), _whole(D, D)],
        out_specs=[_row_blocked(D), _row_blocked(D)],
        out_shape=[jax.ShapeDtypeStruct((N, D), jnp.bfloat16)] * 2,
    )(xb, _bf16(encoder1), _bf16(W_att))

    z1 = pl.pallas_call(
        _attn_body,
        grid=grid,
        compiler_params=pltpu.CompilerParams(dimension_semantics=("parallel",)),
        in_specs=[_whole(8, D), _whole(N, D), _row_blocked(D), _whole(D, D)],
        out_specs=_row_blocked(D),
        out_shape=jax.ShapeDtypeStruct((N, D), jnp.bfloat16),
    )(a_pair, wh, xproj, _bf16(gc1_W))

    z2 = pl.pallas_call(
        _gcn1_body,
        grid=grid,
        compiler_params=pltpu.CompilerParams(dimension_semantics=("parallel",)),
        in_specs=[_row_blocked(N), _whole(N, D), _whole(1, D), _whole(D, D)],
        out_specs=_row_blocked(D),
        out_shape=jax.ShapeDtypeStruct((N, D), jnp.bfloat16),
    )(adjb, z1, gc1_b.reshape(1, D), _bf16(gc2_W))

    z3 = pl.pallas_call(
        _gcn2_body,
        grid=grid,
        compiler_params=pltpu.CompilerParams(dimension_semantics=("parallel",)),
        in_specs=[_row_blocked(N), _whole(N, D), _whole(1, D), _whole(D, C)],
        out_specs=_row_blocked(C),
        out_shape=jax.ShapeDtypeStruct((N, C), jnp.bfloat16),
    )(adjb, z2, gc2_b.reshape(1, D), _bf16(gc3_W))

    out = pl.pallas_call(
        _gcn3_body,
        grid=grid,
        compiler_params=pltpu.CompilerParams(dimension_semantics=("parallel",)),
        in_specs=[_row_blocked(N), _whole(N, C), _whole(1, C)],
        out_specs=_row_blocked(C),
        out_shape=jax.ShapeDtypeStruct((N, C), jnp.float32),
    )(adjb, z3, gc3_b.reshape(1, C))

    return out


# speedup vs baseline: 1.7370x; 1.1872x over previous
import jax
import jax.numpy as jnp
from jax import lax
from jax.experimental import pallas as pl
from jax.experimental.pallas import tpu as pltpu

N = 4096
D = 512
C = 128
BLK = 256
ALPHA = 0.2
_NT = (((1,), (1,)), ((), ()))
_PAR = pltpu.CompilerParams(dimension_semantics=("parallel",))


def _bf16(x):
    return x.astype(jnp.bfloat16)


def _proj_body(x_ref, enc_ref, watt_ref, xproj_ref, wh_ref):
    xp = jnp.dot(_bf16(x_ref[...]), enc_ref[...],
                 preferred_element_type=jnp.float32)
    wh = jnp.dot(_bf16(xp), watt_ref[...], preferred_element_type=jnp.float32)
    xproj_ref[...] = _bf16(xp)
    wh_ref[...] = _bf16(wh)


def _attn_body(a_ref, wh_ref, xproj_ref, w1_ref, z1_ref):
    i = pl.program_id(0)
    wh = wh_ref[...]
    wh_blk = wh_ref[pl.ds(i * BLK, BLK), :]
    a8 = a_ref[...]
    eblk = lax.dot_general(wh_blk, a8, _NT, preferred_element_type=jnp.float32)
    erow = lax.dot_general(a8, wh, _NT, preferred_element_type=jnp.float32)
    e1 = eblk[:, 0:1]
    e2 = erow[1:2, :]
    m2 = jnp.max(e2)
    t = e1 + m2
    m = jnp.where(t >= 0, t, ALPHA * t)
    p = jnp.exp(t - m)
    q = jnp.exp(ALPHA * t - m)
    u = jnp.exp(e2 - m2)
    v = jnp.exp(ALPHA * (e2 - m2))
    s = jnp.where(e1 + e2 > 0, p * u, q * v)
    z = jnp.sum(s, axis=1, keepdims=True)
    acc = jnp.dot(_bf16(s), wh, preferred_element_type=jnp.float32)
    x_ent = xproj_ref[...].astype(jnp.float32) * (acc / z)
    z1_ref[...] = _bf16(
        jnp.dot(_bf16(x_ent), w1_ref[...], preferred_element_type=jnp.float32))


def _gcn1_body(adj_ref, z1_ref, b1_ref, w2_ref, w3_ref, adjb_ref, t1_ref):
    adjb = _bf16(adj_ref[...])
    adjb_ref[...] = adjb
    y1 = jnp.dot(adjb, z1_ref[...], preferred_element_type=jnp.float32)
    y1 = jnp.maximum(y1 + b1_ref[...], 0.0)
    z2 = jnp.dot(_bf16(y1), w2_ref[...], preferred_element_type=jnp.float32)
    t1_ref[...] = _bf16(
        jnp.dot(_bf16(z2), w3_ref[...], preferred_element_type=jnp.float32))


def _tail2_body(adj_ref, t1_ref, b2_ref, w3_ref, z3_ref):
    b2w3 = jnp.dot(b2_ref[...], w3_ref[...],
                   preferred_element_type=jnp.float32)
    y = jnp.dot(adj_ref[...], t1_ref[...], preferred_element_type=jnp.float32)
    z3_ref[...] = _bf16(y + b2w3)


def _gcn3_body(adj_ref, z3_ref, b3_ref, out_ref):
    y3 = jnp.dot(adj_ref[...], z3_ref[...], preferred_element_type=jnp.float32)
    y3 = y3 + b3_ref[...]
    m = jnp.max(y3, axis=1, keepdims=True)
    s = y3 - m
    lse = jnp.log(jnp.sum(jnp.exp(s), axis=1, keepdims=True))
    out_ref[...] = s - lse


def _row_blocked(d):
    return pl.BlockSpec((BLK, d), lambda i: (i, 0))


def _whole(r, c):
    return pl.BlockSpec((r, c), lambda i: (0, 0))


def kernel(x_org, adj, encoder1, W_att, a_att, gc1_W, gc1_b, gc2_W, gc2_b,
           gc3_W, gc3_b):
    grid = (N // BLK,)
    a_pair = jnp.zeros((8, D), jnp.bfloat16).at[0:2].set(
        _bf16(a_att.reshape(2, D)))

    xproj, wh = pl.pallas_call(
        _proj_body,
        grid=grid,
        compiler_params=_PAR,
        in_specs=[_row_blocked(D), _whole(D, D), _whole(D, D)],
        out_specs=[_row_blocked(D), _row_blocked(D)],
        out_shape=[jax.ShapeDtypeStruct((N, D), jnp.bfloat16)] * 2,
    )(x_org, _bf16(encoder1), _bf16(W_att))

    z1 = pl.pallas_call(
        _attn_body,
        grid=grid,
        compiler_params=_PAR,
        in_specs=[_whole(8, D), _whole(N, D), _row_blocked(D), _whole(D, D)],
        out_specs=_row_blocked(D),
        out_shape=jax.ShapeDtypeStruct((N, D), jnp.bfloat16),
    )(a_pair, wh, xproj, _bf16(gc1_W))

    adjb, t1 = pl.pallas_call(
        _gcn1_body,
        grid=grid,
        compiler_params=_PAR,
        in_specs=[_row_blocked(N), _whole(N, D), _whole(1, D), _whole(D, D),
                  _whole(D, C)],
        out_specs=[_row_blocked(N), _row_blocked(C)],
        out_shape=[jax.ShapeDtypeStruct((N, N), jnp.bfloat16),
                   jax.ShapeDtypeStruct((N, C), jnp.bfloat16)],
    )(adj, z1, gc1_b.reshape(1, D), _bf16(gc2_W), _bf16(gc3_W))

    z3 = pl.pallas_call(
        _tail2_body,
        grid=grid,
        compiler_params=_PAR,
        in_specs=[_row_blocked(N), _whole(N, C), _whole(1, D), _whole(D, C)],
        out_specs=_row_blocked(C),
        out_shape=jax.ShapeDtypeStruct((N, C), jnp.bfloat16),
    )(adjb, t1, gc2_b.reshape(1, D), _bf16(gc3_W))

    out = pl.pallas_call(
        _gcn3_body,
        grid=grid,
        compiler_params=_PAR,
        in_specs=[_row_blocked(N), _whole(N, C), _whole(1, C)],
        out_specs=_row_blocked(C),
        out_shape=jax.ShapeDtypeStruct((N, C), jnp.float32),
    )(adjb, z3, gc3_b.reshape(1, C))

    return out


# speedup vs baseline: 1.7939x; 1.0328x over previous
import jax
import jax.numpy as jnp
from jax import lax
from jax.experimental import pallas as pl
from jax.experimental.pallas import tpu as pltpu

N = 4096
D = 512
C = 128
BLK = 256
ALPHA = 0.2
_NT = (((1,), (1,)), ((), ()))
_PAR = pltpu.CompilerParams(dimension_semantics=("parallel",))


def _bf16(x):
    return x.astype(jnp.bfloat16)


def _proj_body(x_ref, enc_ref, watt_ref, xproj_ref, wh_ref):
    xp = jnp.dot(_bf16(x_ref[...]), enc_ref[...],
                 preferred_element_type=jnp.float32)
    wh = jnp.dot(_bf16(xp), watt_ref[...], preferred_element_type=jnp.float32)
    xproj_ref[...] = _bf16(xp)
    wh_ref[...] = _bf16(wh)


def _attn_body(a_ref, wh_ref, xproj_ref, w1_ref, adj_ref, z1_ref, adjb_ref):
    adjb_ref[...] = _bf16(adj_ref[...])
    i = pl.program_id(0)
    wh = wh_ref[...]
    wh_blk = wh_ref[pl.ds(i * BLK, BLK), :]
    a8 = a_ref[...]
    eblk = lax.dot_general(wh_blk, a8, _NT, preferred_element_type=jnp.float32)
    erow = lax.dot_general(a8, wh, _NT, preferred_element_type=jnp.float32)
    e1 = eblk[:, 0:1]
    e2 = erow[1:2, :]
    m2 = jnp.max(e2)
    t = e1 + m2
    m = jnp.where(t >= 0, t, ALPHA * t)
    p = jnp.exp(t - m)
    q = jnp.exp(ALPHA * t - m)
    u = jnp.exp(e2 - m2)
    v = jnp.exp(ALPHA * (e2 - m2))
    s = jnp.where(e1 + e2 > 0, p * u, q * v)
    z = jnp.sum(s, axis=1, keepdims=True)
    acc = jnp.dot(_bf16(s), wh, preferred_element_type=jnp.float32)
    x_ent = xproj_ref[...].astype(jnp.float32) * (acc / z)
    z1_ref[...] = _bf16(
        jnp.dot(_bf16(x_ent), w1_ref[...], preferred_element_type=jnp.float32))


def _gcn1_body(adj_ref, z1_ref, b1_ref, w2_ref, w3_ref, t1_ref):
    y1 = jnp.dot(adj_ref[...], z1_ref[...], preferred_element_type=jnp.float32)
    y1 = jnp.maximum(y1 + b1_ref[...], 0.0)
    z2 = jnp.dot(_bf16(y1), w2_ref[...], preferred_element_type=jnp.float32)
    t1_ref[...] = _bf16(
        jnp.dot(_bf16(z2), w3_ref[...], preferred_element_type=jnp.float32))


def _tail2_body(adj_ref, t1_ref, b2_ref, w3_ref, z3_ref):
    b2w3 = jnp.dot(b2_ref[...], w3_ref[...],
                   preferred_element_type=jnp.float32)
    y = jnp.dot(adj_ref[...], t1_ref[...], preferred_element_type=jnp.float32)
    z3_ref[...] = _bf16(y + b2w3)


def _gcn3_body(adj_ref, z3_ref, b3_ref, out_ref):
    y3 = jnp.dot(adj_ref[...], z3_ref[...], preferred_element_type=jnp.float32)
    y3 = y3 + b3_ref[...]
    m = jnp.max(y3, axis=1, keepdims=True)
    s = y3 - m
    lse = jnp.log(jnp.sum(jnp.exp(s), axis=1, keepdims=True))
    out_ref[...] = s - lse


def _row_blocked(d):
    return pl.BlockSpec((BLK, d), lambda i: (i, 0))


def _whole(r, c):
    return pl.BlockSpec((r, c), lambda i: (0, 0))


def kernel(x_org, adj, encoder1, W_att, a_att, gc1_W, gc1_b, gc2_W, gc2_b,
           gc3_W, gc3_b):
    grid = (N // BLK,)
    a_pair = jnp.zeros((8, D), jnp.bfloat16).at[0:2].set(
        _bf16(a_att.reshape(2, D)))

    xproj, wh = pl.pallas_call(
        _proj_body,
        grid=grid,
        compiler_params=_PAR,
        in_specs=[_row_blocked(D), _whole(D, D), _whole(D, D)],
        out_specs=[_row_blocked(D), _row_blocked(D)],
        out_shape=[jax.ShapeDtypeStruct((N, D), jnp.bfloat16)] * 2,
    )(x_org, _bf16(encoder1), _bf16(W_att))

    z1, adjb = pl.pallas_call(
        _attn_body,
        grid=grid,
        compiler_params=_PAR,
        in_specs=[_whole(8, D), _whole(N, D), _row_blocked(D), _whole(D, D),
                  _row_blocked(N)],
        out_specs=[_row_blocked(D), _row_blocked(N)],
        out_shape=[jax.ShapeDtypeStruct((N, D), jnp.bfloat16),
                   jax.ShapeDtypeStruct((N, N), jnp.bfloat16)],
    )(a_pair, wh, xproj, _bf16(gc1_W), adj)

    t1 = pl.pallas_call(
        _gcn1_body,
        grid=grid,
        compiler_params=_PAR,
        in_specs=[_row_blocked(N), _whole(N, D), _whole(1, D), _whole(D, D),
                  _whole(D, C)],
        out_specs=_row_blocked(C),
        out_shape=jax.ShapeDtypeStruct((N, C), jnp.bfloat16),
    )(adjb, z1, gc1_b.reshape(1, D), _bf16(gc2_W), _bf16(gc3_W))

    z3 = pl.pallas_call(
        _tail2_body,
        grid=grid,
        compiler_params=_PAR,
        in_specs=[_row_blocked(N), _whole(N, C), _whole(1, D), _whole(D, C)],
        out_specs=_row_blocked(C),
        out_shape=jax.ShapeDtypeStruct((N, C), jnp.bfloat16),
    )(adjb, t1, gc2_b.reshape(1, D), _bf16(gc3_W))

    out = pl.pallas_call(
        _gcn3_body,
        grid=grid,
        compiler_params=_PAR,
        in_specs=[_row_blocked(N), _whole(N, C), _whole(1, C)],
        out_specs=_row_blocked(C),
        out_shape=jax.ShapeDtypeStruct((N, C), jnp.float32),
    )(adjb, z3, gc3_b.reshape(1, C))

    return out
